# Initial kernel scaffold; baseline (speedup 1.0000x reference)
import jax
import jax.numpy as jnp
from jax.experimental import pallas as pl


def kernel(x, edge_index, W1, b1, W_self, W_neigh, b2):
    raise NotImplementedError("write your pallas kernel here")


# speedup vs baseline: 7.1793x; 7.1793x over previous
import functools

import jax
import jax.numpy as jnp
from jax import lax
from jax.experimental import pallas as pl
from jax.experimental.pallas import tpu as pltpu
from jax.experimental.pallas import tpu_sc as plsc

NC = 2
NS = 16
NW = NC * NS
CHUNK = 80


def _mesh():
  return plsc.VectorSubcoreMesh(
      core_axis_name="c", subcore_axis_name="s", num_cores=NC,
      num_subcores=NS)


def _deg_call(src, dst, n_pad, e):
  epw = e // NW
  n_chunks = epw // CHUNK
  rows_per_tile = n_pad // NS
  zeros = jnp.zeros((rows_per_tile, 2), jnp.float32)
  ones_src = jnp.tile(jnp.array([[1.0, 0.0]], jnp.float32), (CHUNK, 1))
  ones_dst = jnp.tile(jnp.array([[0.0, 1.0]], jnp.float32), (CHUNK, 1))

  @functools.partial(
      pl.kernel,
      out_type=jax.ShapeDtypeStruct((NC, n_pad, 2), jnp.float32),
      mesh=_mesh(),
      scratch_types=[
          pltpu.VMEM((CHUNK,), jnp.int32),
          pltpu.VMEM((CHUNK,), jnp.int32),
          pltpu.VMEM((CHUNK, 2), jnp.float32),
          pltpu.VMEM((CHUNK, 2), jnp.float32),
          pltpu.VMEM_SHARED((n_pad, 2), jnp.float32),
      ],
  )
  def deg_kernel(src_hbm, dst_hbm, zz_hbm, os_hbm, od_hbm, out_hbm, idx_s,
                 idx_d, ones_s, ones_d, acc):
    cid = lax.axis_index("c")
    sid = lax.axis_index("s")
    wid = sid * NC + cid
    pltpu.sync_copy(zz_hbm, acc.at[pl.ds(sid * rows_per_tile,
                                         rows_per_tile)])
    pltpu.sync_copy(os_hbm, ones_s)
    pltpu.sync_copy(od_hbm, ones_d)
    plsc.subcore_barrier()

    def body(c, _):
      base = wid * epw + c * CHUNK
      pltpu.sync_copy(src_hbm.at[pl.ds(base, CHUNK)], idx_s)
      pltpu.sync_copy(dst_hbm.at[pl.ds(base, CHUNK)], idx_d)
      pltpu.sync_copy(ones_s, acc.at[idx_s], add=True)
      pltpu.sync_copy(ones_d, acc.at[idx_d], add=True)
      return 0

    lax.fori_loop(0, n_chunks, body, 0)
    plsc.subcore_barrier()
    sl = pl.ds(sid * rows_per_tile, rows_per_tile)
    pltpu.sync_copy(acc.at[sl], out_hbm.at[cid, sl, :])

  return deg_kernel(src, dst, zeros, ones_src, ones_dst)


def _scatter_call(table, src, dst, n2, d, e):
  epw = e // NW
  n_chunks = epw // CHUNK
  rows_per_tile = n2 // NS
  zeros = jnp.zeros((rows_per_tile, d), jnp.float32)

  @functools.partial(
      pl.kernel,
      out_type=jax.ShapeDtypeStruct((NC, n2, d), jnp.float32),
      mesh=_mesh(),
      scratch_types=[
          pltpu.VMEM((CHUNK,), jnp.int32),
          pltpu.VMEM((CHUNK,), jnp.int32),
          pltpu.VMEM((CHUNK, d), jnp.float32),
          pltpu.VMEM_SHARED((n2, d), jnp.float32),
          pltpu.SemaphoreType.DMA,
      ],
  )
  def scat_kernel(table_hbm, src_hbm, dst_hbm, zz_hbm, out_hbm, idx_s, idx_d,
                  rows_v, acc, sem):
    cid = lax.axis_index("c")
    sid = lax.axis_index("s")
    wid = sid * NC + cid
    pltpu.sync_copy(zz_hbm, acc.at[pl.ds(sid * rows_per_tile,
                                         rows_per_tile)])
    plsc.subcore_barrier()

    def body(c, _):
      base = wid * epw + c * CHUNK
      pltpu.sync_copy(src_hbm.at[pl.ds(base, CHUNK)], idx_s)
      pltpu.sync_copy(dst_hbm.at[pl.ds(base, CHUNK)], idx_d)
      pltpu.async_copy(table_hbm.at[idx_s], rows_v, sem).wait()
      pltpu.sync_copy(rows_v, acc.at[idx_d], add=True)
      return 0

    lax.fori_loop(0, n_chunks, body, 0)
    plsc.subcore_barrier()
    sl = pl.ds(sid * rows_per_tile, rows_per_tile)
    pltpu.sync_copy(acc.at[sl], out_hbm.at[cid, sl])

  return scat_kernel(table, src, dst, zeros)


_BLK = 1000


def _h_scaled_kernel(x_ref, w1_ref, deg_ref, out_ref):
  norm = lax.rsqrt(jnp.maximum(deg_ref[...], 1.0))
  h = jnp.dot(x_ref[...], w1_ref[...], preferred_element_type=jnp.float32,
              precision=lax.Precision.HIGHEST)
  out_ref[...] = h * norm


def _h1_kernel(aggp_ref, deg_ref, b1_ref, out_ref):
  agg = aggp_ref[0] + aggp_ref[1]
  norm = lax.rsqrt(jnp.maximum(deg_ref[...], 1.0))
  out_ref[...] = jnp.maximum(agg * norm + b1_ref[...], 0.0)


def _out_kernel(h1_ref, nsp_ref, deg_ref, ws_ref, wn_ref, b2_ref, out_ref):
  inv = 1.0 / jnp.maximum(deg_ref[...], 1.0)
  neigh = (nsp_ref[0] + nsp_ref[1]) * inv
  out_ref[...] = (
      jnp.dot(h1_ref[...], ws_ref[...], preferred_element_type=jnp.float32,
              precision=lax.Precision.HIGHEST)
      + jnp.dot(neigh, wn_ref[...], preferred_element_type=jnp.float32,
                precision=lax.Precision.HIGHEST)
      + b2_ref[...])


def kernel(x, edge_index, W1, b1, W_self, W_neigh, b2):
  n, d_in = x.shape
  e = edge_index.shape[1]
  d_hid = W1.shape[1]
  d_out = W_self.shape[1]
  src = edge_index[0]
  dst = edge_index[1]

  n_pad = ((n + 8 * NS - 1) // (8 * NS)) * (8 * NS)
  n2 = n_pad

  deg_parts = _deg_call(src, dst, n_pad, e)
  deg_out_col = (deg_parts[0, :n, 0] + deg_parts[1, :n, 0])[:, None]
  deg_in_col = (deg_parts[0, :n, 1] + deg_parts[1, :n, 1])[:, None]

  grid = n // _BLK
  hs = pl.pallas_call(
      _h_scaled_kernel,
      grid=(grid,),
      in_specs=[
          pl.BlockSpec((_BLK, d_in), lambda i: (i, 0)),
          pl.BlockSpec((d_in, d_hid), lambda i: (0, 0)),
          pl.BlockSpec((_BLK, 1), lambda i: (i, 0)),
      ],
      out_specs=pl.BlockSpec((_BLK, d_hid), lambda i: (i, 0)),
      out_shape=jax.ShapeDtypeStruct((n, d_hid), jnp.float32),
  )(x, W1, deg_out_col)

  agg_parts = _scatter_call(hs, src, dst, n2, d_hid, e)

  h1 = pl.pallas_call(
      _h1_kernel,
      grid=(grid,),
      in_specs=[
          pl.BlockSpec((2, _BLK, d_hid), lambda i: (0, i, 0)),
          pl.BlockSpec((_BLK, 1), lambda i: (i, 0)),
          pl.BlockSpec((d_hid,), lambda i: (0,)),
      ],
      out_specs=pl.BlockSpec((_BLK, d_hid), lambda i: (i, 0)),
      out_shape=jax.ShapeDtypeStruct((n, d_hid), jnp.float32),
  )(agg_parts, deg_in_col, b1)

  ns_parts = _scatter_call(h1, src, dst, n2, d_hid, e)

  out = pl.pallas_call(
      _out_kernel,
      grid=(grid,),
      in_specs=[
          pl.BlockSpec((_BLK, d_hid), lambda i: (i, 0)),
          pl.BlockSpec((2, _BLK, d_hid), lambda i: (0, i, 0)),
          pl.BlockSpec((_BLK, 1), lambda i: (i, 0)),
          pl.BlockSpec((d_hid, d_out), lambda i: (0, 0)),
          pl.BlockSpec((d_hid, d_out), lambda i: (0, 0)),
          pl.BlockSpec((d_out,), lambda i: (0,)),
      ],
      out_specs=pl.BlockSpec((_BLK, d_out), lambda i: (i, 0)),
      out_shape=jax.ShapeDtypeStruct((n, d_out), jnp.float32),
  )(h1, ns_parts, deg_in_col, W_self, W_neigh, b2)

  return out


# speedup vs baseline: 13.2097x; 1.8400x over previous
import functools

import jax
import jax.numpy as jnp
from jax import lax
from jax.experimental import pallas as pl
from jax.experimental.pallas import tpu as pltpu
from jax.experimental.pallas import tpu_sc as plsc

NC = 2
NS = 16
NW = NC * NS
CHUNK = 128
NBUF = 2


def _mesh():
  return plsc.VectorSubcoreMesh(
      core_axis_name="c", subcore_axis_name="s", num_cores=NC,
      num_subcores=NS)


def _deg_call(src, dst, n_pad, e2):
  epw = e2 // NW
  n_chunks = epw // CHUNK
  n_groups = n_chunks // NBUF
  rows_per_tile = n_pad // NS
  zeros = jnp.zeros((rows_per_tile, 2), jnp.float32)
  ones_src = jnp.tile(jnp.array([[1.0, 0.0]], jnp.float32), (CHUNK, 1))
  ones_dst = jnp.tile(jnp.array([[0.0, 1.0]], jnp.float32), (CHUNK, 1))

  @functools.partial(
      pl.kernel,
      out_type=jax.ShapeDtypeStruct((NC, n_pad, 2), jnp.float32),
      mesh=_mesh(),
      scratch_types=[
          [pltpu.VMEM((CHUNK,), jnp.int32) for _ in range(NBUF)],
          [pltpu.VMEM((CHUNK,), jnp.int32) for _ in range(NBUF)],
          pltpu.VMEM((CHUNK, 2), jnp.float32),
          pltpu.VMEM((CHUNK, 2), jnp.float32),
          pltpu.VMEM_SHARED((n_pad, 2), jnp.float32),
          [pltpu.SemaphoreType.DMA for _ in range(NBUF)],
          [pltpu.SemaphoreType.DMA for _ in range(NBUF)],
      ],
  )
  def deg_kernel(src_hbm, dst_hbm, zz_hbm, os_hbm, od_hbm, out_hbm, idx_s,
                 idx_d, ones_s, ones_d, acc, isem, ssem):
    cid = lax.axis_index("c")
    sid = lax.axis_index("s")
    wid = sid * NC + cid
    base0 = wid * epw
    pltpu.sync_copy(zz_hbm, acc.at[pl.ds(sid * rows_per_tile,
                                         rows_per_tile)])
    pltpu.sync_copy(os_hbm, ones_s)
    pltpu.sync_copy(od_hbm, ones_d)
    plsc.subcore_barrier()

    def fire_idx(c, k):
      base = base0 + c * CHUNK
      pltpu.async_copy(src_hbm.at[pl.ds(base, CHUNK)], idx_s[k], isem[k])
      pltpu.async_copy(dst_hbm.at[pl.ds(base, CHUNK)], idx_d[k], isem[k])

    def wait_idx(k):
      pltpu.make_async_copy(src_hbm.at[pl.ds(0, CHUNK)], idx_s[k],
                            isem[k]).wait()
      pltpu.make_async_copy(dst_hbm.at[pl.ds(0, CHUNK)], idx_d[k],
                            isem[k]).wait()

    def group(g, refire):
      for k in range(NBUF):
        wait_idx(k)
        pltpu.async_copy(ones_s, acc.at[idx_s[k]], ssem[k], add=True)
        pltpu.async_copy(ones_d, acc.at[idx_d[k]], ssem[k], add=True)
      for k in range(NBUF):
        pltpu.make_async_copy(ones_s, acc.at[idx_s[k]], ssem[k]).wait()
        pltpu.make_async_copy(ones_d, acc.at[idx_d[k]], ssem[k]).wait()
        if refire:
          fire_idx((g + 1) * NBUF + k, k)

    for k in range(NBUF):
      fire_idx(k, k)
    lax.fori_loop(0, n_groups - 1, lambda g, _: (group(g, True), 0)[1], 0)
    group(n_groups - 1, False)

    plsc.subcore_barrier()
    sl = pl.ds(sid * rows_per_tile, rows_per_tile)
    pltpu.sync_copy(acc.at[sl], out_hbm.at[cid, sl, :])

  return deg_kernel(src, dst, zeros, ones_src, ones_dst)


def _scatter_call(table, src, dst, n2, d, e2):
  epw = e2 // NW
  n_chunks = epw // CHUNK
  n_groups = n_chunks // NBUF
  rows_per_tile = n2 // NS
  zeros = jnp.zeros((rows_per_tile, d), jnp.float32)

  @functools.partial(
      pl.kernel,
      out_type=jax.ShapeDtypeStruct((NC, n2, d), jnp.float32),
      mesh=_mesh(),
      scratch_types=[
          [pltpu.VMEM((CHUNK,), jnp.int32) for _ in range(NBUF)],
          [pltpu.VMEM((CHUNK,), jnp.int32) for _ in range(NBUF)],
          [pltpu.VMEM((CHUNK, d), jnp.float32) for _ in range(NBUF)],
          pltpu.VMEM_SHARED((n2, d), jnp.float32),
          [pltpu.SemaphoreType.DMA for _ in range(NBUF)],
          [pltpu.SemaphoreType.DMA for _ in range(NBUF)],
          [pltpu.SemaphoreType.DMA for _ in range(NBUF)],
      ],
  )
  def scat_kernel(table_hbm, src_hbm, dst_hbm, zz_hbm, out_hbm, idx_s, idx_d,
                  rows_v, acc, isem, gsem, ssem):
    cid = lax.axis_index("c")
    sid = lax.axis_index("s")
    wid = sid * NC + cid
    base0 = wid * epw
    pltpu.sync_copy(zz_hbm, acc.at[pl.ds(sid * rows_per_tile,
                                         rows_per_tile)])
    plsc.subcore_barrier()

    def fire(c, k):
      base = base0 + c * CHUNK
      pltpu.async_copy(src_hbm.at[pl.ds(base, CHUNK)], idx_s[k], isem[k])
      pltpu.async_copy(dst_hbm.at[pl.ds(base, CHUNK)], idx_d[k], isem[k])

    def group(g, refire):
      for k in range(NBUF):
        pltpu.make_async_copy(src_hbm.at[pl.ds(0, CHUNK)], idx_s[k],
                              isem[k]).wait()
        pltpu.make_async_copy(dst_hbm.at[pl.ds(0, CHUNK)], idx_d[k],
                              isem[k]).wait()
        pltpu.async_copy(table_hbm.at[idx_s[k]], rows_v[k], gsem[k])
      for k in range(NBUF):
        pltpu.make_async_copy(table_hbm.at[idx_s[k]], rows_v[k],
                              gsem[k]).wait()
        pltpu.async_copy(rows_v[k], acc.at[idx_d[k]], ssem[k], add=True)
      for k in range(NBUF):
        pltpu.make_async_copy(rows_v[k], acc.at[idx_d[k]], ssem[k]).wait()
        if refire:
          fire((g + 1) * NBUF + k, k)

    for k in range(NBUF):
      fire(k, k)
    lax.fori_loop(0, n_groups - 1, lambda g, _: (group(g, True), 0)[1], 0)
    group(n_groups - 1, False)

    plsc.subcore_barrier()
    sl = pl.ds(sid * rows_per_tile, rows_per_tile)
    pltpu.sync_copy(acc.at[sl], out_hbm.at[cid, sl])

  return scat_kernel(table, src, dst, zeros)


_BLK = 1000


def _h_scaled_kernel(x_ref, w1_ref, deg_ref, out_ref):
  norm = lax.rsqrt(jnp.maximum(deg_ref[...], 1.0))
  h = jnp.dot(x_ref[...], w1_ref[...], preferred_element_type=jnp.float32,
              precision=lax.Precision.HIGHEST)
  out_ref[...] = h * norm


def _h1_kernel(aggp_ref, deg_ref, b1_ref, out_ref):
  agg = aggp_ref[0] + aggp_ref[1]
  norm = lax.rsqrt(jnp.maximum(deg_ref[...], 1.0))
  out_ref[...] = jnp.maximum(agg * norm + b1_ref[...], 0.0)


def _out_kernel(h1_ref, nsp_ref, deg_ref, ws_ref, wn_ref, b2_ref, out_ref):
  inv = 1.0 / jnp.maximum(deg_ref[...], 1.0)
  neigh = (nsp_ref[0] + nsp_ref[1]) * inv
  out_ref[...] = (
      jnp.dot(h1_ref[...], ws_ref[...], preferred_element_type=jnp.float32,
              precision=lax.Precision.HIGHEST)
      + jnp.dot(neigh, wn_ref[...], preferred_element_type=jnp.float32,
                precision=lax.Precision.HIGHEST)
      + b2_ref[...])


def kernel(x, edge_index, W1, b1, W_self, W_neigh, b2):
  n, d_in = x.shape
  e = edge_index.shape[1]
  d_hid = W1.shape[1]
  d_out = W_self.shape[1]
  src = edge_index[0]
  dst = edge_index[1]

  n_pad = ((n + 8 * NS) // (8 * NS)) * (8 * NS)
  n2 = n_pad

  step = NW * CHUNK * NBUF
  e2 = ((e + step - 1) // step) * step
  pad = e2 - e
  pad_lo = jnp.arange(pad, dtype=jnp.int32) % n
  pad_hi = n + jnp.arange(pad, dtype=jnp.int32) % (n_pad - n)
  src_deg = jnp.concatenate([src, pad_hi])
  src_gat = jnp.concatenate([src, pad_lo])
  dst_p = jnp.concatenate([dst, pad_hi])

  deg_parts = _deg_call(src_deg, dst_p, n_pad, e2)
  deg_out_col = (deg_parts[0, :n, 0] + deg_parts[1, :n, 0])[:, None]
  deg_in_col = (deg_parts[0, :n, 1] + deg_parts[1, :n, 1])[:, None]

  grid = n // _BLK
  hs = pl.pallas_call(
      _h_scaled_kernel,
      grid=(grid,),
      in_specs=[
          pl.BlockSpec((_BLK, d_in), lambda i: (i, 0)),
          pl.BlockSpec((d_in, d_hid), lambda i: (0, 0)),
          pl.BlockSpec((_BLK, 1), lambda i: (i, 0)),
      ],
      out_specs=pl.BlockSpec((_BLK, d_hid), lambda i: (i, 0)),
      out_shape=jax.ShapeDtypeStruct((n, d_hid), jnp.float32),
  )(x, W1, deg_out_col)

  agg_parts = _scatter_call(hs, src_gat, dst_p, n2, d_hid, e2)

  h1 = pl.pallas_call(
      _h1_kernel,
      grid=(grid,),
      in_specs=[
          pl.BlockSpec((2, _BLK, d_hid), lambda i: (0, i, 0)),
          pl.BlockSpec((_BLK, 1), lambda i: (i, 0)),
          pl.BlockSpec((d_hid,), lambda i: (0,)),
      ],
      out_specs=pl.BlockSpec((_BLK, d_hid), lambda i: (i, 0)),
      out_shape=jax.ShapeDtypeStruct((n, d_hid), jnp.float32),
  )(agg_parts, deg_in_col, b1)

  ns_parts = _scatter_call(h1, src_gat, dst_p, n2, d_hid, e2)

  out = pl.pallas_call(
      _out_kernel,
      grid=(grid,),
      in_specs=[
          pl.BlockSpec((_BLK, d_hid), lambda i: (i, 0)),
          pl.BlockSpec((2, _BLK, d_hid), lambda i: (0, i, 0)),
          pl.BlockSpec((_BLK, 1), lambda i: (i, 0)),
          pl.BlockSpec((d_hid, d_out), lambda i: (0, 0)),
          pl.BlockSpec((d_hid, d_out), lambda i: (0, 0)),
          pl.BlockSpec((d_out,), lambda i: (0,)),
      ],
      out_specs=pl.BlockSpec((_BLK, d_out), lambda i: (i, 0)),
      out_shape=jax.ShapeDtypeStruct((n, d_out), jnp.float32),
  )(h1, ns_parts, deg_in_col, W_self, W_neigh, b2)

  return out


# speedup vs baseline: 14.8402x; 1.1234x over previous
import functools

import jax
import jax.numpy as jnp
from jax import lax
from jax.experimental import pallas as pl
from jax.experimental.pallas import tpu as pltpu
from jax.experimental.pallas import tpu_sc as plsc

NC = 2
NS = 16
NW = NC * NS
CHUNK = 64
NBUF = 4


def _mesh():
  return plsc.VectorSubcoreMesh(
      core_axis_name="c", subcore_axis_name="s", num_cores=NC,
      num_subcores=NS)


def _deg_call(src, dst, n_pad, e2):
  epw = e2 // NW
  n_chunks = epw // CHUNK
  n_groups = n_chunks // NBUF
  rows_per_tile = n_pad // NS
  zeros = jnp.zeros((rows_per_tile, 2), jnp.float32)
  ones_src = jnp.tile(jnp.array([[1.0, 0.0]], jnp.float32), (CHUNK, 1))
  ones_dst = jnp.tile(jnp.array([[0.0, 1.0]], jnp.float32), (CHUNK, 1))

  @functools.partial(
      pl.kernel,
      out_type=jax.ShapeDtypeStruct((NC, n_pad, 2), jnp.float32),
      mesh=_mesh(),
      scratch_types=[
          [pltpu.VMEM((CHUNK,), jnp.int32) for _ in range(NBUF)],
          [pltpu.VMEM((CHUNK,), jnp.int32) for _ in range(NBUF)],
          pltpu.VMEM((CHUNK, 2), jnp.float32),
          pltpu.VMEM((CHUNK, 2), jnp.float32),
          pltpu.VMEM_SHARED((n_pad, 2), jnp.float32),
          [pltpu.SemaphoreType.DMA for _ in range(NBUF)],
          [pltpu.SemaphoreType.DMA for _ in range(NBUF)],
      ],
  )
  def deg_kernel(src_hbm, dst_hbm, zz_hbm, os_hbm, od_hbm, out_hbm, idx_s,
                 idx_d, ones_s, ones_d, acc, isem, ssem):
    cid = lax.axis_index("c")
    sid = lax.axis_index("s")
    wid = sid * NC + cid
    base0 = wid * epw
    pltpu.sync_copy(zz_hbm, acc.at[pl.ds(sid * rows_per_tile,
                                         rows_per_tile)])
    pltpu.sync_copy(os_hbm, ones_s)
    pltpu.sync_copy(od_hbm, ones_d)
    plsc.subcore_barrier()

    def fire_idx(c, k):
      base = base0 + c * CHUNK
      pltpu.async_copy(src_hbm.at[pl.ds(base, CHUNK)], idx_s[k], isem[k])
      pltpu.async_copy(dst_hbm.at[pl.ds(base, CHUNK)], idx_d[k], isem[k])

    def wait_idx(k):
      pltpu.make_async_copy(src_hbm.at[pl.ds(0, CHUNK)], idx_s[k],
                            isem[k]).wait()
      pltpu.make_async_copy(dst_hbm.at[pl.ds(0, CHUNK)], idx_d[k],
                            isem[k]).wait()

    def group(g, refire):
      for k in range(NBUF):
        wait_idx(k)
        pltpu.async_copy(ones_s, acc.at[idx_s[k]], ssem[k], add=True)
        pltpu.async_copy(ones_d, acc.at[idx_d[k]], ssem[k], add=True)
      for k in range(NBUF):
        pltpu.make_async_copy(ones_s, acc.at[idx_s[k]], ssem[k]).wait()
        pltpu.make_async_copy(ones_d, acc.at[idx_d[k]], ssem[k]).wait()
        if refire:
          fire_idx((g + 1) * NBUF + k, k)

    for k in range(NBUF):
      fire_idx(k, k)
    lax.fori_loop(0, n_groups - 1, lambda g, _: (group(g, True), 0)[1], 0)
    group(n_groups - 1, False)

    plsc.subcore_barrier()
    sl = pl.ds(sid * rows_per_tile, rows_per_tile)
    pltpu.sync_copy(acc.at[sl], out_hbm.at[cid, sl, :])

  return deg_kernel(src, dst, zeros, ones_src, ones_dst)


def _scatter_call(table, src, dst, n2, d, e2):
  epw = e2 // NW
  n_chunks = epw // CHUNK
  n_groups = n_chunks // NBUF
  rows_per_tile = n2 // NS
  zeros = jnp.zeros((rows_per_tile, d), jnp.float32)

  @functools.partial(
      pl.kernel,
      out_type=jax.ShapeDtypeStruct((NC, n2, d), jnp.float32),
      mesh=_mesh(),
      scratch_types=[
          [pltpu.VMEM((CHUNK,), jnp.int32) for _ in range(NBUF)],
          [pltpu.VMEM((CHUNK,), jnp.int32) for _ in range(NBUF)],
          [pltpu.VMEM((CHUNK, d), jnp.float32) for _ in range(NBUF)],
          pltpu.VMEM_SHARED((n2, d), jnp.float32),
          [pltpu.SemaphoreType.DMA for _ in range(NBUF)],
          [pltpu.SemaphoreType.DMA for _ in range(NBUF)],
          [pltpu.SemaphoreType.DMA for _ in range(NBUF)],
      ],
  )
  def scat_kernel(table_hbm, src_hbm, dst_hbm, zz_hbm, out_hbm, idx_s, idx_d,
                  rows_v, acc, isem, gsem, ssem):
    cid = lax.axis_index("c")
    sid = lax.axis_index("s")
    wid = sid * NC + cid
    base0 = wid * epw
    pltpu.sync_copy(zz_hbm, acc.at[pl.ds(sid * rows_per_tile,
                                         rows_per_tile)])
    plsc.subcore_barrier()

    def fire(c, k):
      base = base0 + c * CHUNK
      pltpu.async_copy(src_hbm.at[pl.ds(base, CHUNK)], idx_s[k], isem[k])
      pltpu.async_copy(dst_hbm.at[pl.ds(base, CHUNK)], idx_d[k], isem[k])

    def group(g, refire):
      for k in range(NBUF):
        pltpu.make_async_copy(src_hbm.at[pl.ds(0, CHUNK)], idx_s[k],
                              isem[k]).wait()
        pltpu.make_async_copy(dst_hbm.at[pl.ds(0, CHUNK)], idx_d[k],
                              isem[k]).wait()
        pltpu.async_copy(table_hbm.at[idx_s[k]], rows_v[k], gsem[k])
      for k in range(NBUF):
        pltpu.make_async_copy(table_hbm.at[idx_s[k]], rows_v[k],
                              gsem[k]).wait()
        pltpu.async_copy(rows_v[k], acc.at[idx_d[k]], ssem[k], add=True)
      for k in range(NBUF):
        pltpu.make_async_copy(rows_v[k], acc.at[idx_d[k]], ssem[k]).wait()
        if refire:
          fire((g + 1) * NBUF + k, k)

    for k in range(NBUF):
      fire(k, k)
    lax.fori_loop(0, n_groups - 1, lambda g, _: (group(g, True), 0)[1], 0)
    group(n_groups - 1, False)

    plsc.subcore_barrier()
    sl = pl.ds(sid * rows_per_tile, rows_per_tile)
    pltpu.sync_copy(acc.at[sl], out_hbm.at[cid, sl])

  return scat_kernel(table, src, dst, zeros)


_BLK = 1000


def _h_scaled_kernel(x_ref, w1_ref, deg_ref, out_ref):
  norm = lax.rsqrt(jnp.maximum(deg_ref[...], 1.0))
  h = jnp.dot(x_ref[...], w1_ref[...], preferred_element_type=jnp.float32,
              precision=lax.Precision.HIGHEST)
  out_ref[...] = h * norm


def _h1_kernel(aggp_ref, deg_ref, b1_ref, out_ref):
  agg = aggp_ref[0] + aggp_ref[1]
  norm = lax.rsqrt(jnp.maximum(deg_ref[...], 1.0))
  out_ref[...] = jnp.maximum(agg * norm + b1_ref[...], 0.0)


def _out_kernel(h1_ref, nsp_ref, deg_ref, ws_ref, wn_ref, b2_ref, out_ref):
  inv = 1.0 / jnp.maximum(deg_ref[...], 1.0)
  neigh = (nsp_ref[0] + nsp_ref[1]) * inv
  out_ref[...] = (
      jnp.dot(h1_ref[...], ws_ref[...], preferred_element_type=jnp.float32,
              precision=lax.Precision.HIGHEST)
      + jnp.dot(neigh, wn_ref[...], preferred_element_type=jnp.float32,
                precision=lax.Precision.HIGHEST)
      + b2_ref[...])


def kernel(x, edge_index, W1, b1, W_self, W_neigh, b2):
  n, d_in = x.shape
  e = edge_index.shape[1]
  d_hid = W1.shape[1]
  d_out = W_self.shape[1]
  src = edge_index[0]
  dst = edge_index[1]

  n_pad = ((n + 8 * NS) // (8 * NS)) * (8 * NS)
  n2 = n_pad

  step = NW * CHUNK * NBUF
  e2 = ((e + step - 1) // step) * step
  pad = e2 - e
  pad_lo = jnp.arange(pad, dtype=jnp.int32) % n
  pad_hi = n + jnp.arange(pad, dtype=jnp.int32) % (n_pad - n)
  src_deg = jnp.concatenate([src, pad_hi])
  src_gat = jnp.concatenate([src, pad_lo])
  dst_p = jnp.concatenate([dst, pad_hi])

  deg_parts = _deg_call(src_deg, dst_p, n_pad, e2)
  deg_out_col = (deg_parts[0, :n, 0] + deg_parts[1, :n, 0])[:, None]
  deg_in_col = (deg_parts[0, :n, 1] + deg_parts[1, :n, 1])[:, None]

  grid = n // _BLK
  hs = pl.pallas_call(
      _h_scaled_kernel,
      grid=(grid,),
      in_specs=[
          pl.BlockSpec((_BLK, d_in), lambda i: (i, 0)),
          pl.BlockSpec((d_in, d_hid), lambda i: (0, 0)),
          pl.BlockSpec((_BLK, 1), lambda i: (i, 0)),
      ],
      out_specs=pl.BlockSpec((_BLK, d_hid), lambda i: (i, 0)),
      out_shape=jax.ShapeDtypeStruct((n, d_hid), jnp.float32),
  )(x, W1, deg_out_col)

  agg_parts = _scatter_call(hs, src_gat, dst_p, n2, d_hid, e2)

  h1 = pl.pallas_call(
      _h1_kernel,
      grid=(grid,),
      in_specs=[
          pl.BlockSpec((2, _BLK, d_hid), lambda i: (0, i, 0)),
          pl.BlockSpec((_BLK, 1), lambda i: (i, 0)),
          pl.BlockSpec((d_hid,), lambda i: (0,)),
      ],
      out_specs=pl.BlockSpec((_BLK, d_hid), lambda i: (i, 0)),
      out_shape=jax.ShapeDtypeStruct((n, d_hid), jnp.float32),
  )(agg_parts, deg_in_col, b1)

  ns_parts = _scatter_call(h1, src_gat, dst_p, n2, d_hid, e2)

  out = pl.pallas_call(
      _out_kernel,
      grid=(grid,),
      in_specs=[
          pl.BlockSpec((_BLK, d_hid), lambda i: (i, 0)),
          pl.BlockSpec((2, _BLK, d_hid), lambda i: (0, i, 0)),
          pl.BlockSpec((_BLK, 1), lambda i: (i, 0)),
          pl.BlockSpec((d_hid, d_out), lambda i: (0, 0)),
          pl.BlockSpec((d_hid, d_out), lambda i: (0, 0)),
          pl.BlockSpec((d_out,), lambda i: (0,)),
      ],
      out_specs=pl.BlockSpec((_BLK, d_out), lambda i: (i, 0)),
      out_shape=jax.ShapeDtypeStruct((n, d_out), jnp.float32),
  )(h1, ns_parts, deg_in_col, W_self, W_neigh, b2)

  return out
